# baseline (speedup 1.0000x reference)
import jax
import jax.numpy as jnp
from jax.experimental import pallas as pl
from jax.experimental.pallas import tpu as pltpu

_N = 62
_NP = 64
_F = 16
_H = 128
_NB = 16
_HI = jax.lax.Precision.HIGHEST


def _fused_kernel(x_ref, ew_ref, linW_ref, linb_ref, crow_ref, c2b_ref,
                  fcW_ref, fcb_ref, out_ref, a2_scr, cmat_scr, bias_scr):
    f32 = jnp.float32
    bf16 = jnp.bfloat16
    bb = x_ref.shape[0]

    @pl.when(pl.program_id(0) == 0)
    def _preamble():
        ew = ew_ref[...]
        rows = [ew[:, i * (i + 1) // 2: i * (i + 1) // 2 + _N]
                for i in range(_N)]
        lraw = jnp.concatenate(rows, axis=0)
        ri = jax.lax.broadcasted_iota(jnp.int32, (_N, _N), 0)
        ci = jax.lax.broadcasted_iota(jnp.int32, (_N, _N), 1)
        eyef = (ri == ci).astype(f32)
        low = jnp.where(ci <= ri, lraw, 0.0)
        lowt = jax.lax.dot_general(eyef, low, (((1,), (1,)), ((), ())),
                                   preferred_element_type=f32, precision=_HI)
        wmat = low + lowt - low * eyef
        absw = jnp.abs(wmat)
        discol = jax.lax.rsqrt(jnp.sum(absw, axis=1, keepdims=True) + 1.0)
        disrow = jax.lax.rsqrt(jnp.sum(absw, axis=0, keepdims=True) + 1.0)
        amat = discol * (wmat + eyef) * disrow
        a2 = jnp.dot(amat, amat, preferred_element_type=f32, precision=_HI)
        a2_scr[...] = jnp.concatenate(
            [a2, jnp.zeros((_N, _NP - _N), f32)], axis=1)
        crow = crow_ref[...]
        crow64 = jnp.concatenate(
            [crow, jnp.zeros((1, _NP - _N), f32)], axis=1)
        ctile = jnp.concatenate([crow64] * bb, axis=1)
        rowi = jax.lax.broadcasted_iota(jnp.int32, (bb, bb * _NP), 0)
        coli = jax.lax.broadcasted_iota(jnp.int32, (bb, bb * _NP), 1)
        lo = rowi * _NP
        blockmask = (coli >= lo) & (coli < lo + _NP)
        cmat_scr[...] = jnp.where(
            blockmask, jnp.broadcast_to(ctile, (bb, bb * _NP)), 0.0
        ).astype(bf16)
        bias_scr[...] = (jnp.sum(crow, keepdims=True) * linb_ref[...]
                         + c2b_ref[...])

    x3 = x_ref[...]
    xt = jax.lax.transpose(x3, (0, 2, 1))
    xt2 = xt.reshape(bb * _F, _N)
    hp = jnp.dot(xt2, a2_scr[...], preferred_element_type=f32,
                 precision=_HI)
    hpb = hp.astype(bf16)
    hpt = jax.lax.transpose(hpb.reshape(bb, _F, _NP), (0, 2, 1))
    h2d = hpt.reshape(bb * _NP, _F)
    sb = jnp.dot(h2d, linW_ref[...].astype(bf16),
                 preferred_element_type=f32).astype(bf16)
    conv = jnp.dot(cmat_scr[...], sb, preferred_element_type=f32)
    z = jnp.maximum(conv + bias_scr[...], 0.0)
    out_ref[...] = jnp.dot(z.astype(bf16), fcW_ref[...].astype(bf16),
                           preferred_element_type=f32) + fcb_ref[...]


def kernel(x, edge_weight, lin_W, lin_b, conv2_W, conv2_b, fc_W, fc_b, edge_idx):
    B, N, F = x.shape
    H = lin_W.shape[1]
    C = fc_W.shape[1]
    ew2 = edge_weight.reshape(1, -1)
    linb2 = lin_b.reshape(1, H)
    crow = conv2_W.reshape(1, N)
    c2b2 = conv2_b.reshape(1, 1)
    fcb2 = fc_b.reshape(1, C)
    bb = B // _NB
    small = lambda shape: pl.BlockSpec(shape, lambda i: tuple(0 for _ in shape))
    return pl.pallas_call(
        _fused_kernel,
        grid=(_NB,),
        in_specs=[
            pl.BlockSpec((bb, N, F), lambda i: (i, 0, 0)),
            small(ew2.shape), small(lin_W.shape), small(linb2.shape),
            small(crow.shape), small(c2b2.shape), small(fc_W.shape),
            small(fcb2.shape),
        ],
        out_specs=pl.BlockSpec((bb, C), lambda i: (i, 0)),
        out_shape=jax.ShapeDtypeStruct((B, C), jnp.float32),
        scratch_shapes=[pltpu.VMEM((_N, _NP), jnp.float32),
                        pltpu.VMEM((bb, bb * _NP), jnp.bfloat16),
                        pltpu.VMEM((1, _H), jnp.float32)],
    )(x, ew2, lin_W, linb2, crow, c2b2, fc_W, fcb2)


# speedup vs baseline: 1.0650x; 1.0650x over previous
import jax
import jax.numpy as jnp
from jax.experimental import pallas as pl
from jax.experimental.pallas import tpu as pltpu

_N = 62
_NP = 64
_F = 16
_H = 128
_NB = 4
_HI = jax.lax.Precision.HIGHEST


def _fused_kernel(x_ref, ew_ref, linW_ref, linb_ref, crow_ref, c2b_ref,
                  fcW_ref, fcb_ref, out_ref, a2_scr, cmat_scr, bias_scr):
    f32 = jnp.float32
    bf16 = jnp.bfloat16
    bb = x_ref.shape[0]

    @pl.when(pl.program_id(0) == 0)
    def _preamble():
        ew = ew_ref[...]
        rows = [ew[:, i * (i + 1) // 2: i * (i + 1) // 2 + _N]
                for i in range(_N)]
        lraw = jnp.concatenate(rows, axis=0)
        ri = jax.lax.broadcasted_iota(jnp.int32, (_N, _N), 0)
        ci = jax.lax.broadcasted_iota(jnp.int32, (_N, _N), 1)
        eyef = (ri == ci).astype(f32)
        low = jnp.where(ci <= ri, lraw, 0.0)
        lowt = jax.lax.dot_general(eyef, low, (((1,), (1,)), ((), ())),
                                   preferred_element_type=f32, precision=_HI)
        wmat = low + lowt - low * eyef
        absw = jnp.abs(wmat)
        discol = jax.lax.rsqrt(jnp.sum(absw, axis=1, keepdims=True) + 1.0)
        disrow = jax.lax.rsqrt(jnp.sum(absw, axis=0, keepdims=True) + 1.0)
        amat = discol * (wmat + eyef) * disrow
        a2 = jnp.dot(amat, amat, preferred_element_type=f32, precision=_HI)
        a2_scr[...] = jnp.concatenate(
            [a2, jnp.zeros((_N, _NP - _N), f32)], axis=1)
        crow = crow_ref[...]
        crow64 = jnp.concatenate(
            [crow, jnp.zeros((1, _NP - _N), f32)], axis=1)
        ctile = jnp.concatenate([crow64] * bb, axis=1)
        rowi = jax.lax.broadcasted_iota(jnp.int32, (bb, bb * _NP), 0)
        coli = jax.lax.broadcasted_iota(jnp.int32, (bb, bb * _NP), 1)
        lo = rowi * _NP
        blockmask = (coli >= lo) & (coli < lo + _NP)
        cmat_scr[...] = jnp.where(
            blockmask, jnp.broadcast_to(ctile, (bb, bb * _NP)), 0.0
        ).astype(bf16)
        bias_scr[...] = (jnp.sum(crow, keepdims=True) * linb_ref[...]
                         + c2b_ref[...])

    x3 = x_ref[...]
    xt = jax.lax.transpose(x3, (0, 2, 1))
    xt2 = xt.reshape(bb * _F, _N)
    hp = jnp.dot(xt2, a2_scr[...], preferred_element_type=f32,
                 precision=_HI)
    hpb = hp.astype(bf16)
    hpt = jax.lax.transpose(hpb.reshape(bb, _F, _NP), (0, 2, 1))
    h2d = hpt.reshape(bb * _NP, _F)
    sb = jnp.dot(h2d, linW_ref[...].astype(bf16),
                 preferred_element_type=f32).astype(bf16)
    conv = jnp.dot(cmat_scr[...], sb, preferred_element_type=f32)
    z = jnp.maximum(conv + bias_scr[...], 0.0)
    out_ref[...] = jnp.dot(z.astype(bf16), fcW_ref[...].astype(bf16),
                           preferred_element_type=f32) + fcb_ref[...]


def kernel(x, edge_weight, lin_W, lin_b, conv2_W, conv2_b, fc_W, fc_b, edge_idx):
    B, N, F = x.shape
    H = lin_W.shape[1]
    C = fc_W.shape[1]
    ew2 = edge_weight.reshape(1, -1)
    linb2 = lin_b.reshape(1, H)
    crow = conv2_W.reshape(1, N)
    c2b2 = conv2_b.reshape(1, 1)
    fcb2 = fc_b.reshape(1, C)
    bb = B // _NB
    small = lambda shape: pl.BlockSpec(shape, lambda i: tuple(0 for _ in shape))
    return pl.pallas_call(
        _fused_kernel,
        grid=(_NB,),
        in_specs=[
            pl.BlockSpec((bb, N, F), lambda i: (i, 0, 0)),
            small(ew2.shape), small(lin_W.shape), small(linb2.shape),
            small(crow.shape), small(c2b2.shape), small(fc_W.shape),
            small(fcb2.shape),
        ],
        out_specs=pl.BlockSpec((bb, C), lambda i: (i, 0)),
        out_shape=jax.ShapeDtypeStruct((B, C), jnp.float32),
        scratch_shapes=[pltpu.VMEM((_N, _NP), jnp.float32),
                        pltpu.VMEM((bb, bb * _NP), jnp.bfloat16),
                        pltpu.VMEM((1, _H), jnp.float32)],
    )(x, ew2, lin_W, linb2, crow, c2b2, fc_W, fcb2)


# speedup vs baseline: 1.3485x; 1.2662x over previous
import jax
import jax.numpy as jnp
from jax.experimental import pallas as pl
from jax.experimental.pallas import tpu as pltpu

_N = 62
_NP = 64
_F = 16
_H = 128
_NB = 8
_HI = jax.lax.Precision.HIGHEST


def _fused_kernel(x_ref, ew_ref, linW_ref, linb_ref, crow_ref, c2b_ref,
                  fcW_ref, fcb_ref, out_ref, a2_scr, cmat_scr, bias_scr):
    f32 = jnp.float32
    bf16 = jnp.bfloat16
    bb = x_ref.shape[0]

    @pl.when(pl.program_id(0) == 0)
    def _preamble():
        ew = ew_ref[...]
        rows = [ew[:, i * (i + 1) // 2: i * (i + 1) // 2 + _N]
                for i in range(_N)]
        lraw = jnp.concatenate(rows, axis=0)
        ri = jax.lax.broadcasted_iota(jnp.int32, (_N, _N), 0)
        ci = jax.lax.broadcasted_iota(jnp.int32, (_N, _N), 1)
        eyef = (ri == ci).astype(f32)
        low = jnp.where(ci <= ri, lraw, 0.0)
        lowt = jax.lax.dot_general(eyef, low, (((1,), (1,)), ((), ())),
                                   preferred_element_type=f32, precision=_HI)
        wmat = low + lowt - low * eyef
        absw = jnp.abs(wmat)
        discol = jax.lax.rsqrt(jnp.sum(absw, axis=1, keepdims=True) + 1.0)
        disrow = jax.lax.rsqrt(jnp.sum(absw, axis=0, keepdims=True) + 1.0)
        amat = discol * (wmat + eyef) * disrow
        a2 = jnp.dot(amat, amat, preferred_element_type=f32, precision=_HI)
        a2_scr[...] = jnp.concatenate(
            [a2, jnp.zeros((_N, _NP - _N), f32)], axis=1)
        crow = crow_ref[...]
        crow64 = jnp.concatenate(
            [crow, jnp.zeros((1, _NP - _N), f32)], axis=1)
        ctile = jnp.concatenate([crow64] * bb, axis=1)
        rowi = jax.lax.broadcasted_iota(jnp.int32, (bb, bb * _NP), 0)
        coli = jax.lax.broadcasted_iota(jnp.int32, (bb, bb * _NP), 1)
        lo = rowi * _NP
        blockmask = (coli >= lo) & (coli < lo + _NP)
        cmat_scr[...] = jnp.where(
            blockmask, jnp.broadcast_to(ctile, (bb, bb * _NP)), 0.0
        ).astype(bf16)
        bias_scr[...] = (jnp.sum(crow, keepdims=True) * linb_ref[...]
                         + c2b_ref[...])

    x3 = x_ref[...].reshape(bb, _N, _F)
    xt = jax.lax.transpose(x3, (0, 2, 1))
    xt2 = xt.reshape(bb * _F, _N)
    hp = jnp.dot(xt2, a2_scr[...], preferred_element_type=f32,
                 precision=_HI)
    hpb = hp.astype(bf16)
    hpt = jax.lax.transpose(hpb.reshape(bb, _F, _NP), (0, 2, 1))
    h2d = hpt.reshape(bb * _NP, _F)
    sb = jnp.dot(h2d, linW_ref[...].astype(bf16),
                 preferred_element_type=f32).astype(bf16)
    conv = jnp.dot(cmat_scr[...], sb, preferred_element_type=f32)
    z = jnp.maximum(conv + bias_scr[...], 0.0)
    out_ref[...] = jnp.dot(z.astype(bf16), fcW_ref[...].astype(bf16),
                           preferred_element_type=f32) + fcb_ref[...]


def kernel(x, edge_weight, lin_W, lin_b, conv2_W, conv2_b, fc_W, fc_b, edge_idx):
    B, N, F = x.shape
    H = lin_W.shape[1]
    C = fc_W.shape[1]
    ew2 = edge_weight.reshape(1, -1)
    linb2 = lin_b.reshape(1, H)
    crow = conv2_W.reshape(1, N)
    c2b2 = conv2_b.reshape(1, 1)
    fcb2 = fc_b.reshape(1, C)
    bb = B // _NB
    small = lambda shape: pl.BlockSpec(shape, lambda i: tuple(0 for _ in shape))
    return pl.pallas_call(
        _fused_kernel,
        grid=(_NB,),
        in_specs=[
            pl.BlockSpec((bb, N * F), lambda i: (i, 0)),
            small(ew2.shape), small(lin_W.shape), small(linb2.shape),
            small(crow.shape), small(c2b2.shape), small(fc_W.shape),
            small(fcb2.shape),
        ],
        out_specs=pl.BlockSpec((bb, C), lambda i: (i, 0)),
        out_shape=jax.ShapeDtypeStruct((B, C), jnp.float32),
        scratch_shapes=[pltpu.VMEM((_N, _NP), jnp.float32),
                        pltpu.VMEM((bb, bb * _NP), jnp.bfloat16),
                        pltpu.VMEM((1, _H), jnp.float32)],
    )(x.reshape(B, N * F), ew2, lin_W, linb2, crow, c2b2, fc_W, fcb2)


# speedup vs baseline: 1.5979x; 1.1850x over previous
import jax
import jax.numpy as jnp
from jax.experimental import pallas as pl
from jax.experimental.pallas import tpu as pltpu

_N = 62
_NP = 64
_F = 16
_H = 128
_NB = 8
_HI = jax.lax.Precision.HIGHEST


def _fused_kernel(x_ref, ew_ref, linW_ref, linb_ref, crow_ref, c2b_ref,
                  fcW_ref, fcb_ref, out_ref, a2_scr, cmat_scr, bias_scr):
    f32 = jnp.float32
    bf16 = jnp.bfloat16
    bb = x_ref.shape[0] // _F

    @pl.when(pl.program_id(0) == 0)
    def _preamble():
        ew = ew_ref[...]
        rows = [ew[:, i * (i + 1) // 2: i * (i + 1) // 2 + _N]
                for i in range(_N)]
        lraw = jnp.concatenate(rows, axis=0)
        ri = jax.lax.broadcasted_iota(jnp.int32, (_N, _N), 0)
        ci = jax.lax.broadcasted_iota(jnp.int32, (_N, _N), 1)
        eyef = (ri == ci).astype(f32)
        low = jnp.where(ci <= ri, lraw, 0.0)
        lowt = jax.lax.dot_general(eyef, low, (((1,), (1,)), ((), ())),
                                   preferred_element_type=f32, precision=_HI)
        wmat = low + lowt - low * eyef
        absw = jnp.abs(wmat)
        discol = jax.lax.rsqrt(jnp.sum(absw, axis=1, keepdims=True) + 1.0)
        disrow = jax.lax.rsqrt(jnp.sum(absw, axis=0, keepdims=True) + 1.0)
        amat = discol * (wmat + eyef) * disrow
        a2 = jnp.dot(amat, amat, preferred_element_type=f32, precision=_HI)
        a2_scr[...] = jnp.concatenate(
            [a2, jnp.zeros((_N, _NP - _N), f32)], axis=1)
        crow = crow_ref[...]
        crow64 = jnp.concatenate(
            [crow, jnp.zeros((1, _NP - _N), f32)], axis=1)
        ctile = jnp.concatenate([crow64] * bb, axis=1)
        rowi = jax.lax.broadcasted_iota(jnp.int32, (bb, bb * _NP), 0)
        coli = jax.lax.broadcasted_iota(jnp.int32, (bb, bb * _NP), 1)
        lo = rowi * _NP
        blockmask = (coli >= lo) & (coli < lo + _NP)
        cmat_scr[...] = jnp.where(
            blockmask, jnp.broadcast_to(ctile, (bb, bb * _NP)), 0.0
        ).astype(bf16)
        bias_scr[...] = (jnp.sum(crow, keepdims=True) * linb_ref[...]
                         + c2b_ref[...])

    xt2 = x_ref[...]
    hp = jnp.dot(xt2, a2_scr[...], preferred_element_type=f32,
                 precision=_HI)
    hpb = hp.astype(bf16)
    hpt = jax.lax.transpose(hpb.reshape(bb, _F, _NP), (0, 2, 1))
    h2d = hpt.reshape(bb * _NP, _F)
    sb = jnp.dot(h2d, linW_ref[...].astype(bf16),
                 preferred_element_type=f32).astype(bf16)
    conv = jnp.dot(cmat_scr[...], sb, preferred_element_type=f32)
    z = jnp.maximum(conv + bias_scr[...], 0.0)
    out_ref[...] = jnp.dot(z.astype(bf16), fcW_ref[...].astype(bf16),
                           preferred_element_type=f32) + fcb_ref[...]


def kernel(x, edge_weight, lin_W, lin_b, conv2_W, conv2_b, fc_W, fc_b, edge_idx):
    B, N, F = x.shape
    H = lin_W.shape[1]
    C = fc_W.shape[1]
    ew2 = edge_weight.reshape(1, -1)
    linb2 = lin_b.reshape(1, H)
    crow = conv2_W.reshape(1, N)
    c2b2 = conv2_b.reshape(1, 1)
    fcb2 = fc_b.reshape(1, C)
    bb = B // _NB
    small = lambda shape: pl.BlockSpec(shape, lambda i: tuple(0 for _ in shape))
    return pl.pallas_call(
        _fused_kernel,
        grid=(_NB,),
        in_specs=[
            pl.BlockSpec((bb * F, N), lambda i: (i, 0)),
            small(ew2.shape), small(lin_W.shape), small(linb2.shape),
            small(crow.shape), small(c2b2.shape), small(fc_W.shape),
            small(fcb2.shape),
        ],
        out_specs=pl.BlockSpec((bb, C), lambda i: (i, 0)),
        out_shape=jax.ShapeDtypeStruct((B, C), jnp.float32),
        scratch_shapes=[pltpu.VMEM((_N, _NP), jnp.float32),
                        pltpu.VMEM((bb, bb * _NP), jnp.bfloat16),
                        pltpu.VMEM((1, _H), jnp.float32)],
    )(x.transpose(0, 2, 1).reshape(B * F, N),
      ew2, lin_W, linb2, crow, c2b2, fc_W, fcb2)
